# speedup vs baseline: 1.2296x; 1.2296x over previous
import functools

import jax
import jax.numpy as jnp
from jax.experimental import pallas as pl
from jax.experimental.pallas import tpu as pltpu

N = 10000
D = 256
H = 512
O = 256

_ROW_BLK = 1000


def _linear_body(x_ref, w_ref, b_ref, add_ref, o_ref, *, relu):
    acc = jnp.dot(x_ref[...], w_ref[...], preferred_element_type=jnp.float32)
    acc = acc + b_ref[...]
    if add_ref is not None:
        acc = acc + add_ref[...]
    if relu:
        acc = jnp.maximum(acc, 0.0)
    o_ref[...] = acc


def _linear(x, w, b, add=None, relu=False):
    n, k = x.shape
    f = w.shape[1]
    grid = (n // _ROW_BLK,)
    in_specs = [
        pl.BlockSpec((_ROW_BLK, k), lambda i: (i, 0)),
        pl.BlockSpec((k, f), lambda i: (0, 0)),
        pl.BlockSpec((1, f), lambda i: (0, 0)),
    ]
    args = [x, w, b.reshape(1, f)]
    if add is not None:
        in_specs.append(pl.BlockSpec((_ROW_BLK, f), lambda i: (i, 0)))
        args.append(add)
        body = functools.partial(_linear_body, relu=relu)
    else:
        body = lambda x_ref, w_ref, b_ref, o_ref: _linear_body(
            x_ref, w_ref, b_ref, None, o_ref, relu=relu)
    return pl.pallas_call(
        body,
        grid=grid,
        in_specs=in_specs,
        out_specs=pl.BlockSpec((_ROW_BLK, f), lambda i: (i, 0)),
        out_shape=jax.ShapeDtypeStruct((n, f), jnp.float32),
    )(*args)


def _agg(x, src, dst, w, inv_deg):
    msg = x[src] * w[:, None]
    s = jax.ops.segment_sum(msg, dst, num_segments=N)
    return s * inv_deg[:, None]


def _inv_deg(dst):
    deg = jax.ops.segment_sum(jnp.ones((dst.shape[0],), jnp.float32), dst,
                              num_segments=N)
    return 1.0 / jnp.clip(deg, 1.0, None)


def kernel(x_acoustic, x_word, edge_index_sim_tic, weight_sim_tic, Ws1_sim_tic, Wn1_sim_tic, b1_sim_tic, Ws2_sim_tic, Wn2_sim_tic, b2_sim_tic, edge_index_sim_w, weight_sim_w, Ws1_sim_w, Wn1_sim_w, b1_sim_w, Ws2_sim_w, Wn2_sim_w, b2_sim_w, edge_index_related_to, weight_related_to, Ws1_related_to, Wn1_related_to, b1_related_to, Ws2_related_to, Wn2_related_to, b2_related_to):
    src_t, dst_t = edge_index_sim_tic[0], edge_index_sim_tic[1]
    src_w, dst_w = edge_index_sim_w[0], edge_index_sim_w[1]
    src_r, dst_r = edge_index_related_to[0], edge_index_related_to[1]

    inv_t = _inv_deg(dst_t)
    inv_w = _inv_deg(dst_w)
    inv_r = _inv_deg(dst_r)

    hn_t = _agg(x_acoustic, src_t, dst_t, weight_sim_tic, inv_t)
    hn_w = _agg(x_word, src_w, dst_w, weight_sim_w, inv_w)
    hn_r = _agg(x_acoustic, src_r, dst_r, weight_related_to, inv_r)

    h_ac = _linear(jnp.concatenate([x_acoustic, hn_t], axis=1),
                   jnp.concatenate([Ws1_sim_tic, Wn1_sim_tic], axis=0),
                   b1_sim_tic, relu=True)
    h_w = _linear(jnp.concatenate([x_word, hn_w, hn_r], axis=1),
                  jnp.concatenate([0.5 * (Ws1_sim_w + Ws1_related_to),
                                   0.5 * Wn1_sim_w,
                                   0.5 * Wn1_related_to], axis=0),
                  0.5 * (b1_sim_w + b1_related_to), relu=True)

    y_t = _linear(h_ac, Wn2_sim_tic, jnp.zeros((O,), jnp.float32))
    y_w = _linear(h_w, Wn2_sim_w, jnp.zeros((O,), jnp.float32))
    y_r = _linear(h_ac, Wn2_related_to, jnp.zeros((O,), jnp.float32))

    hn2_t = _agg(y_t, src_t, dst_t, weight_sim_tic, inv_t)
    hn2_w = _agg(y_w, src_w, dst_w, weight_sim_w, inv_w)
    hn2_r = _agg(y_r, src_r, dst_r, weight_related_to, inv_r)

    o_ac = _linear(h_ac, Ws2_sim_tic, b2_sim_tic, add=hn2_t)
    o_w = _linear(h_w, 0.5 * (Ws2_sim_w + Ws2_related_to),
                  0.5 * (b2_sim_w + b2_related_to),
                  add=0.5 * (hn2_w + hn2_r))
    return (o_ac, o_w)


# speedup vs baseline: 1.8371x; 1.4940x over previous
import functools

import jax
import jax.numpy as jnp
from jax import lax
from jax.experimental import pallas as pl
from jax.experimental.pallas import tpu as pltpu
from jax.experimental.pallas import tpu_sc as plsc

N = 10000
E = 160000
D = 256
H = 512
O = 256

_NC = 2
_NT = 16
_HALF = N // _NC
_ET = E // _NT
_STG = 640
_NB = -(-_ET // 128)
_ZROW = 320
_DR = 48

_ROW_BLK = 1000


def _linear_body(x_ref, w_ref, b_ref, add_ref, o_ref, *, relu):
    acc = jnp.dot(x_ref[...], w_ref[...], preferred_element_type=jnp.float32)
    acc = acc + b_ref[...]
    if add_ref is not None:
        acc = acc + add_ref[...]
    if relu:
        acc = jnp.maximum(acc, 0.0)
    o_ref[...] = acc


def _linear(x, w, b, add=None, relu=False):
    n, k = x.shape
    f = w.shape[1]
    grid = (n // _ROW_BLK,)
    in_specs = [
        pl.BlockSpec((_ROW_BLK, k), lambda i: (i, 0)),
        pl.BlockSpec((k, f), lambda i: (0, 0)),
        pl.BlockSpec((1, f), lambda i: (0, 0)),
    ]
    args = [x, w, b.reshape(1, f)]
    if add is not None:
        in_specs.append(pl.BlockSpec((_ROW_BLK, f), lambda i: (i, 0)))
        args.append(add)
        body = functools.partial(_linear_body, relu=relu)
    else:
        body = lambda x_ref, w_ref, b_ref, o_ref: _linear_body(
            x_ref, w_ref, b_ref, None, o_ref, relu=relu)
    return pl.pallas_call(
        body,
        grid=grid,
        in_specs=in_specs,
        out_specs=pl.BlockSpec((_ROW_BLK, f), lambda i: (i, 0)),
        out_shape=jax.ShapeDtypeStruct((n, f), jnp.float32),
    )(*args)


_NW = _NC * _NT
_TROW = 320
_SUB = 160
_NPAD = _NW * _TROW
_BS = 64
_RING = 32
_SSTG = 1280
_NGRP = 314
_HW = 128
_AR = _SUB + 8


def _scan_body(ei0, w0, ei1, w1, ei2, w2,
               ls_hbm, ld_hbm, lw_hbm, cnt_hbm,
               sd0, sd1, ew0, ew1, rs0, rd0, rw0, rs1, rd1, rw1,
               cntbuf, sem0, sem1):
    c = lax.axis_index("c")
    s = lax.axis_index("s")
    g = c * _NT + s
    base = g * _TROW
    rings = ((rs0, rd0, rw0), (rs1, rd1, rw1))

    for rel, (ei, w_hbm) in enumerate(((ei0, w0), (ei1, w1), (ei2, w2))):
        def issue(h, sd, ew, sem):
            pltpu.async_copy(ei.at[:, pl.ds(h * _SSTG, _SSTG)], sd, sem)
            pltpu.async_copy(w_hbm.at[pl.ds(h * _SSTG, _SSTG)], ew, sem)

        def wait(sd, ew, sem):
            pltpu.make_async_copy(ei.at[:, pl.ds(0, _SSTG)], sd, sem).wait()
            pltpu.make_async_copy(w_hbm.at[pl.ds(0, _SSTG)], ew, sem).wait()

        def filt_stage(sd, ew, carry):
            def filt(i, carry):
                c0, c1, f0, f1 = carry
                sv = sd[0, pl.ds(i * 16, 16)]
                dv = sd[1, pl.ds(i * 16, 16)]
                wv = ew[pl.ds(i * 16, 16)]
                ld = dv - base
                cs = []
                for sub, cc in ((0, c0), (1, c1)):
                    lds = ld - sub * _SUB
                    m = (lds >= 0) & (lds < _SUB)
                    mi = jnp.where(m, 1, 0)
                    pos = cc + plsc.cumsum(mi) - 1
                    row = lax.shift_right_logical(pos, 6) & (_RING - 1)
                    col = pos & (_BS - 1)
                    rs, rd, rw = rings[sub]
                    plsc.store_scatter(rs, [row, col], sv, mask=m)
                    plsc.store_scatter(rd, [row, col], lds, mask=m)
                    plsc.store_scatter(rw, [row, col], wv, mask=m)
                    cs.append(cc + jnp.sum(mi))
                return cs[0], cs[1], f0, f1
            def filt_pl(i, carry):
                return filt(i, carry)
            return plsc.parallel_loop(0, _SSTG // 16, unroll=4,
                                      carry=carry)(filt_pl)

        def flush(sub, cc, fl):
            rs, rd, rw = rings[sub]

            def grp(j8, _):
                rr = (j8 & 3) * 8
                pltpu.sync_copy(rs.at[pl.ds(rr, 8)],
                                ls_hbm.at[rel, sub, g, j8])
                pltpu.sync_copy(rd.at[pl.ds(rr, 8)],
                                ld_hbm.at[rel, sub, g, j8])
                pltpu.sync_copy(rw.at[pl.ds(rr, 8)],
                                lw_hbm.at[rel, sub, g, j8])
                return 0
            lax.fori_loop(fl, cc, grp, 0)
            return cc

        def stage_pair(hh, carry):
            h0 = 2 * hh
            wait(sd0, ew0, sem0)
            issue(h0 + 1, sd1, ew1, sem1)
            carry = filt_stage(sd0, ew0, carry)
            c0, c1, f0, f1 = carry
            f0 = flush(0, lax.shift_right_logical(c0, 9), f0)
            f1 = flush(1, lax.shift_right_logical(c1, 9), f1)
            wait(sd1, ew1, sem1)
            issue(h0 + 2, sd0, ew0, sem0)
            carry = filt_stage(sd1, ew1, (c0, c1, f0, f1))
            c0, c1, f0, f1 = carry
            f0 = flush(0, lax.shift_right_logical(c0, 9), f0)
            f1 = flush(1, lax.shift_right_logical(c1, 9), f1)
            return c0, c1, f0, f1

        issue(0, sd0, ew0, sem0)
        nst = E // _SSTG
        carry = lax.fori_loop(0, (nst - 1) // 2, stage_pair,
                              (jnp.int32(0), jnp.int32(0),
                               jnp.int32(0), jnp.int32(0)))
        wait(sd0, ew0, sem0)
        c0, c1, f0, f1 = filt_stage(sd0, ew0, carry)

        for sub, cc, fl in ((0, c0, f0), (1, c1, f1)):
            ng = lax.shift_right_logical(cc + 511, 9)
            flush(sub, ng, fl)
            cntbuf[0, pl.ds(0, 16)] = jnp.full((16,), cc, jnp.int32)
            pltpu.sync_copy(cntbuf, cnt_hbm.at[rel, sub, g])


def _agg_body(x0l, x0h, x1l, x1h, x2l, x2h,
              ls_hbm, ld_hbm, lw_hbm, cnt_hbm, out_hbm,
              rbs, rbd, rbw, idxa, idxb, idxc, wsa, wsb, wsc,
              dsa, dsb, dsc, gbufa, gbufb, gbufc, acc, cntv,
              sema, semb, semc):
    c = lax.axis_index("c")
    s = lax.axis_index("s")
    g = c * _NT + s
    lane = lax.iota(jnp.int32, 16)
    zf = jnp.zeros((16,), jnp.float32)
    zc = jnp.zeros((16,), jnp.int32)
    onef = jnp.ones((16,), jnp.float32)
    cols = [lane + 16 * k for k in range(_HW // 16)]
    bufs = ((idxa, wsa, dsa, gbufa, sema), (idxb, wsb, dsb, gbufb, semb),
            (idxc, wsc, dsc, gbufc, semc))

    for rel, xlh in enumerate(((x0l, x0h), (x1l, x1h), (x2l, x2h))):
        def subbody(sub, _):
            pltpu.sync_copy(cnt_hbm.at[rel, sub, g], cntv)
            cnt = jnp.sum(jnp.where(lane == 0, cntv[0, pl.ds(0, 16)], zc))
            nb = lax.shift_right_logical(cnt + _BS - 1, 6)

            def readback(t):
                half = lax.shift_right_logical(t, 3) & 1
                grp = lax.shift_right_logical(t, 3)
                pltpu.sync_copy(ls_hbm.at[rel, sub, g, grp],
                                rbs.at[half])
                pltpu.sync_copy(ld_hbm.at[rel, sub, g, grp],
                                rbd.at[half])
                pltpu.sync_copy(lw_hbm.at[rel, sub, g, grp],
                                rbw.at[half])

            for hp, x_hbm in enumerate(xlh):
                @plsc.parallel_loop(0, _AR, unroll=4)
                def _(r):
                    for k in range(_HW // 16):
                        acc[r, pl.ds(16 * k, 16)] = zf

                def sanitize_and_issue(t, buf):
                    idx, wsn, dsn, gbuf, sem = buf
                    ja = lax.shift_right_logical(t, 3) & 1
                    jb = t & 7
                    for k in range(_BS // 16):
                        posv = t * _BS + 16 * k + lane
                        valid = posv < cnt
                        sv = rbs[ja, jb, pl.ds(16 * k, 16)]
                        dvv = rbd[ja, jb, pl.ds(16 * k, 16)]
                        wvv = rbw[ja, jb, pl.ds(16 * k, 16)]
                        idx[pl.ds(16 * k, 16)] = jnp.where(
                            valid & (sv >= 0) & (sv < N), sv, 0)
                        dsn[pl.ds(16 * k, 16)] = jnp.where(
                            valid & (dvv >= 0) & (dvv < _SUB), dvv, 0)
                        wsn[pl.ds(16 * k, 16)] = jnp.where(valid, wvv, zf)
                    pltpu.async_copy(x_hbm.at[idx], gbuf, sem)

                def process(j, buf):
                    idx, wsn, dsn, gbuf, sem = buf

                    def srow(r, _):
                        rv = jnp.full((16,), r, jnp.int32)
                        wv = plsc.load_gather(wsn, [rv])
                        dv = plsc.load_gather(dsn, [rv])
                        for k in range(_HW // 16):
                            plsc.addupdate_scatter(
                                acc, [dv, cols[k]],
                                gbuf[r, pl.ds(16 * k, 16)] * wv)
                        if hp == 0:
                            real = (j * _BS + r) < cnt
                            drow = _SUB + lax.shift_right_logical(dv, 7)
                            plsc.addupdate_scatter(
                                acc, [drow, dv & 127], onef,
                                mask=(lane == 0) & real)
                        return 0
                    lax.fori_loop(0, _BS, srow, 0, unroll=4)

                @pl.when(nb > 0)
                def _():
                    readback(0)
                    sanitize_and_issue(0, bufs[0])

                @pl.when(nb > 1)
                def _():
                    sanitize_and_issue(1, bufs[1])

                @pl.when(nb > 2)
                def _():
                    sanitize_and_issue(2, bufs[2])

                def triple(jj, _):
                    for b in range(3):
                        j = 3 * jj + b
                        buf = bufs[b]

                        @pl.when(j < nb)
                        def _():
                            idx, wsn, dsn, gbuf, sem = buf
                            pltpu.make_async_copy(
                                x_hbm.at[pl.ds(0, _BS)], gbuf, sem).wait()
                            process(j, buf)
                            t = j + 3

                            @pl.when(t < nb)
                            def _():
                                @pl.when((t & 7) == 0)
                                def _():
                                    readback(t)
                                sanitize_and_issue(t, buf)
                    return 0
                lax.fori_loop(0, (nb + 2) // 3, triple, 0)

                pltpu.sync_copy(acc, out_hbm.at[rel, sub, hp, g])
            return 0
        lax.fori_loop(0, 2, subbody, 0)


def _make_scan():
    mesh = plsc.VectorSubcoreMesh(core_axis_name="c", subcore_axis_name="s",
                                  num_cores=_NC, num_subcores=_NT)
    return pl.kernel(
        _scan_body,
        out_type=(jax.ShapeDtypeStruct((3, 2, _NW, _NGRP, 8, _BS), jnp.int32),
                  jax.ShapeDtypeStruct((3, 2, _NW, _NGRP, 8, _BS), jnp.int32),
                  jax.ShapeDtypeStruct((3, 2, _NW, _NGRP, 8, _BS),
                                       jnp.float32),
                  jax.ShapeDtypeStruct((3, 2, _NW, 1, 16), jnp.int32)),
        mesh=mesh,
        compiler_params=pltpu.CompilerParams(needs_layout_passes=False),
        scratch_types=[
            pltpu.VMEM((2, _SSTG), jnp.int32),
            pltpu.VMEM((2, _SSTG), jnp.int32),
            pltpu.VMEM((_SSTG,), jnp.float32),
            pltpu.VMEM((_SSTG,), jnp.float32),
            pltpu.VMEM((_RING, _BS), jnp.int32),
            pltpu.VMEM((_RING, _BS), jnp.int32),
            pltpu.VMEM((_RING, _BS), jnp.float32),
            pltpu.VMEM((_RING, _BS), jnp.int32),
            pltpu.VMEM((_RING, _BS), jnp.int32),
            pltpu.VMEM((_RING, _BS), jnp.float32),
            pltpu.VMEM((1, 16), jnp.int32),
            pltpu.SemaphoreType.DMA,
            pltpu.SemaphoreType.DMA,
        ],
    )


def _make_aggB():
    mesh = plsc.VectorSubcoreMesh(core_axis_name="c", subcore_axis_name="s",
                                  num_cores=_NC, num_subcores=_NT)
    return pl.kernel(
        _agg_body,
        out_type=jax.ShapeDtypeStruct((3, 2, 2, _NW, _AR, _HW), jnp.float32),
        mesh=mesh,
        compiler_params=pltpu.CompilerParams(needs_layout_passes=False),
        scratch_types=[
            pltpu.VMEM((2, 8, _BS), jnp.int32),
            pltpu.VMEM((2, 8, _BS), jnp.int32),
            pltpu.VMEM((2, 8, _BS), jnp.float32),
            pltpu.VMEM((_BS,), jnp.int32),
            pltpu.VMEM((_BS,), jnp.int32),
            pltpu.VMEM((_BS,), jnp.int32),
            pltpu.VMEM((_BS,), jnp.float32),
            pltpu.VMEM((_BS,), jnp.float32),
            pltpu.VMEM((_BS,), jnp.float32),
            pltpu.VMEM((_BS,), jnp.int32),
            pltpu.VMEM((_BS,), jnp.int32),
            pltpu.VMEM((_BS,), jnp.int32),
            pltpu.VMEM((_BS, _HW), jnp.float32),
            pltpu.VMEM((_BS, _HW), jnp.float32),
            pltpu.VMEM((_BS, _HW), jnp.float32),
            pltpu.VMEM((_AR, _HW), jnp.float32),
            pltpu.VMEM((1, 16), jnp.int32),
            pltpu.SemaphoreType.DMA,
            pltpu.SemaphoreType.DMA,
            pltpu.SemaphoreType.DMA,
        ],
    )


_scan = _make_scan()
_aggB = _make_aggB()


def _route(ei_list, w_list):
    return _scan(ei_list[0], w_list[0], ei_list[1], w_list[1],
                 ei_list[2], w_list[2])


def _sc_agg3(x_list, lists):
    ls, ld, lw, cnts = lists
    xs = []
    for x in x_list:
        xs += [x[:, :_HW], x[:, _HW:]]
    out = _aggB(*xs, ls, ld, lw, cnts)
    aggs, degs = [], []
    for r in range(3):
        a = out[r, :, :, :, :_SUB, :]
        a = jnp.transpose(a, (2, 0, 3, 1, 4))
        aggs.append(a.reshape(_NPAD, D)[:N])
        d = out[r, :, 0, :, _SUB:_SUB + 2, :]
        d = jnp.transpose(d, (1, 0, 2, 3))
        d = d.reshape(_NW, 2, 256)[:, :, :_SUB]
        degs.append(d.reshape(_NPAD)[:N])
    return aggs, degs


def kernel(x_acoustic, x_word, edge_index_sim_tic, weight_sim_tic, Ws1_sim_tic, Wn1_sim_tic, b1_sim_tic, Ws2_sim_tic, Wn2_sim_tic, b2_sim_tic, edge_index_sim_w, weight_sim_w, Ws1_sim_w, Wn1_sim_w, b1_sim_w, Ws2_sim_w, Wn2_sim_w, b2_sim_w, edge_index_related_to, weight_related_to, Ws1_related_to, Wn1_related_to, b1_related_to, Ws2_related_to, Wn2_related_to, b2_related_to):
    lists = _route((edge_index_sim_tic, edge_index_sim_w,
                    edge_index_related_to),
                   (weight_sim_tic, weight_sim_w, weight_related_to))
    (a1_t, a1_w, a1_r), (deg_t, deg_w, deg_r) = _sc_agg3(
        (x_acoustic, x_word, x_acoustic), lists)

    inv_t = 1.0 / jnp.clip(deg_t, 1.0, None)
    inv_w = 1.0 / jnp.clip(deg_w, 1.0, None)
    inv_r = 1.0 / jnp.clip(deg_r, 1.0, None)

    hn_t = a1_t * inv_t[:, None]
    hn_w = a1_w * inv_w[:, None]
    hn_r = a1_r * inv_r[:, None]

    h_ac = _linear(jnp.concatenate([x_acoustic, hn_t], axis=1),
                   jnp.concatenate([Ws1_sim_tic, Wn1_sim_tic], axis=0),
                   b1_sim_tic, relu=True)
    h_w = _linear(jnp.concatenate([x_word, hn_w, hn_r], axis=1),
                  jnp.concatenate([0.5 * (Ws1_sim_w + Ws1_related_to),
                                   0.5 * Wn1_sim_w,
                                   0.5 * Wn1_related_to], axis=0),
                  0.5 * (b1_sim_w + b1_related_to), relu=True)

    y_t = _linear(h_ac, Wn2_sim_tic, jnp.zeros((O,), jnp.float32))
    y_w = _linear(h_w, Wn2_sim_w, jnp.zeros((O,), jnp.float32))
    y_r = _linear(h_ac, Wn2_related_to, jnp.zeros((O,), jnp.float32))

    (a2_t, a2_w, a2_r), _ = _sc_agg3((y_t, y_w, y_r), lists)

    hn2_t = a2_t * inv_t[:, None]
    hn2_w = a2_w * inv_w[:, None]
    hn2_r = a2_r * inv_r[:, None]

    o_ac = _linear(h_ac, Ws2_sim_tic, b2_sim_tic, add=hn2_t)
    o_w = _linear(h_w, 0.5 * (Ws2_sim_w + Ws2_related_to),
                  0.5 * (b2_sim_w + b2_related_to),
                  add=0.5 * (hn2_w + hn2_r))
    return (o_ac, o_w)


# speedup vs baseline: 1.8457x; 1.0047x over previous
import functools

import jax
import jax.numpy as jnp
from jax import lax
from jax.experimental import pallas as pl
from jax.experimental.pallas import tpu as pltpu
from jax.experimental.pallas import tpu_sc as plsc

N = 10000
E = 160000
D = 256
H = 512
O = 256

_NC = 2
_NT = 16
_HALF = N // _NC
_ET = E // _NT
_STG = 640
_NB = -(-_ET // 128)
_ZROW = 320
_DR = 48

_ROW_BLK = 1000


def _linear_body(x_ref, w_ref, b_ref, add_ref, o_ref, *, relu):
    acc = jnp.dot(x_ref[...], w_ref[...], preferred_element_type=jnp.float32)
    acc = acc + b_ref[...]
    if add_ref is not None:
        acc = acc + add_ref[...]
    if relu:
        acc = jnp.maximum(acc, 0.0)
    o_ref[...] = acc


def _linear(x, w, b, add=None, relu=False):
    n, k = x.shape
    f = w.shape[1]
    grid = (n // _ROW_BLK,)
    in_specs = [
        pl.BlockSpec((_ROW_BLK, k), lambda i: (i, 0)),
        pl.BlockSpec((k, f), lambda i: (0, 0)),
        pl.BlockSpec((1, f), lambda i: (0, 0)),
    ]
    args = [x, w, b.reshape(1, f)]
    if add is not None:
        in_specs.append(pl.BlockSpec((_ROW_BLK, f), lambda i: (i, 0)))
        args.append(add)
        body = functools.partial(_linear_body, relu=relu)
    else:
        body = lambda x_ref, w_ref, b_ref, o_ref: _linear_body(
            x_ref, w_ref, b_ref, None, o_ref, relu=relu)
    return pl.pallas_call(
        body,
        grid=grid,
        in_specs=in_specs,
        out_specs=pl.BlockSpec((_ROW_BLK, f), lambda i: (i, 0)),
        out_shape=jax.ShapeDtypeStruct((n, f), jnp.float32),
    )(*args)


_NW = _NC * _NT
_TROW = 320
_SUB = 160
_NPAD = _NW * _TROW
_BS = 64
_RING = 32
_SSTG = 1280
_NGRP = 314
_HW = 128
_AR = _SUB + 8


def _scan_body(ei0, w0, ei1, w1, ei2, w2,
               ls_hbm, ld_hbm, lw_hbm, cnt_hbm,
               sd0, sd1, ew0, ew1, rs0, rd0, rw0, rs1, rd1, rw1,
               cntbuf, sem0, sem1):
    c = lax.axis_index("c")
    s = lax.axis_index("s")
    g = c * _NT + s
    base = g * _TROW
    rings = ((rs0, rd0, rw0), (rs1, rd1, rw1))

    for rel, (ei, w_hbm) in enumerate(((ei0, w0), (ei1, w1), (ei2, w2))):
        def issue(h, sd, ew, sem):
            pltpu.async_copy(ei.at[:, pl.ds(h * _SSTG, _SSTG)], sd, sem)
            pltpu.async_copy(w_hbm.at[pl.ds(h * _SSTG, _SSTG)], ew, sem)

        def wait(sd, ew, sem):
            pltpu.make_async_copy(ei.at[:, pl.ds(0, _SSTG)], sd, sem).wait()
            pltpu.make_async_copy(w_hbm.at[pl.ds(0, _SSTG)], ew, sem).wait()

        def filt_stage(sd, ew, carry):
            def filt(i, carry):
                c0, c1, f0, f1 = carry
                sv = sd[0, pl.ds(i * 16, 16)]
                dv = sd[1, pl.ds(i * 16, 16)]
                wv = ew[pl.ds(i * 16, 16)]
                ld = dv - base
                cs = []
                for sub, cc in ((0, c0), (1, c1)):
                    lds = ld - sub * _SUB
                    m = (lds >= 0) & (lds < _SUB)
                    mi = jnp.where(m, 1, 0)
                    pos = cc + plsc.cumsum(mi) - 1
                    row = lax.shift_right_logical(pos, 6) & (_RING - 1)
                    col = pos & (_BS - 1)
                    rs, rd, rw = rings[sub]
                    plsc.store_scatter(rs, [row, col], sv, mask=m)
                    plsc.store_scatter(rd, [row, col], lds, mask=m)
                    plsc.store_scatter(rw, [row, col], wv, mask=m)
                    cs.append(cc + jnp.sum(mi))
                return cs[0], cs[1], f0, f1
            def filt_pl(i, carry):
                return filt(i, carry)
            return plsc.parallel_loop(0, _SSTG // 16, unroll=4,
                                      carry=carry)(filt_pl)

        def flush(sub, cc, fl):
            rs, rd, rw = rings[sub]

            def grp(j8, _):
                rr = (j8 & 3) * 8
                pltpu.sync_copy(rs.at[pl.ds(rr, 8)],
                                ls_hbm.at[rel, sub, g, j8])
                pltpu.sync_copy(rd.at[pl.ds(rr, 8)],
                                ld_hbm.at[rel, sub, g, j8])
                pltpu.sync_copy(rw.at[pl.ds(rr, 8)],
                                lw_hbm.at[rel, sub, g, j8])
                return 0
            lax.fori_loop(fl, cc, grp, 0)
            return cc

        def stage_pair(hh, carry):
            h0 = 2 * hh
            wait(sd0, ew0, sem0)
            issue(h0 + 1, sd1, ew1, sem1)
            carry = filt_stage(sd0, ew0, carry)
            c0, c1, f0, f1 = carry
            f0 = flush(0, lax.shift_right_logical(c0, 9), f0)
            f1 = flush(1, lax.shift_right_logical(c1, 9), f1)
            wait(sd1, ew1, sem1)
            issue(h0 + 2, sd0, ew0, sem0)
            carry = filt_stage(sd1, ew1, (c0, c1, f0, f1))
            c0, c1, f0, f1 = carry
            f0 = flush(0, lax.shift_right_logical(c0, 9), f0)
            f1 = flush(1, lax.shift_right_logical(c1, 9), f1)
            return c0, c1, f0, f1

        issue(0, sd0, ew0, sem0)
        nst = E // _SSTG
        carry = lax.fori_loop(0, (nst - 1) // 2, stage_pair,
                              (jnp.int32(0), jnp.int32(0),
                               jnp.int32(0), jnp.int32(0)))
        wait(sd0, ew0, sem0)
        c0, c1, f0, f1 = filt_stage(sd0, ew0, carry)

        for sub, cc, fl in ((0, c0, f0), (1, c1, f1)):
            ng = lax.shift_right_logical(cc + 511, 9)
            flush(sub, ng, fl)
            cntbuf[0, pl.ds(0, 16)] = jnp.full((16,), cc, jnp.int32)
            pltpu.sync_copy(cntbuf, cnt_hbm.at[rel, sub, g])


def _agg_body(x0l, x0h, x1l, x1h, x2l, x2h,
              ls_hbm, ld_hbm, lw_hbm, cnt_hbm, out_hbm,
              rbs, rbd, rbw, idxa, idxb, idxc, wsa, wsb, wsc,
              dsa, dsb, dsc, gbufa, gbufb, gbufc, acc, cntv,
              sema, semb, semc):
    c = lax.axis_index("c")
    s = lax.axis_index("s")
    g = c * _NT + s
    lane = lax.iota(jnp.int32, 16)
    zf = jnp.zeros((16,), jnp.float32)
    zc = jnp.zeros((16,), jnp.int32)
    onef = jnp.ones((16,), jnp.float32)
    cols = [lane + 16 * k for k in range(_HW // 16)]
    bufs = ((idxa, wsa, dsa, gbufa, sema), (idxb, wsb, dsb, gbufb, semb),
            (idxc, wsc, dsc, gbufc, semc))

    for rel, xlh in enumerate(((x0l, x0h), (x1l, x1h), (x2l, x2h))):
        def subbody(sub, _):
            pltpu.sync_copy(cnt_hbm.at[rel, sub, g], cntv)
            cnt = jnp.sum(jnp.where(lane == 0, cntv[0, pl.ds(0, 16)], zc))
            nb = lax.shift_right_logical(cnt + _BS - 1, 6)

            def readback(t):
                half = lax.shift_right_logical(t, 3) & 1
                grp = lax.shift_right_logical(t, 3)
                pltpu.sync_copy(ls_hbm.at[rel, sub, g, grp],
                                rbs.at[half])
                pltpu.sync_copy(ld_hbm.at[rel, sub, g, grp],
                                rbd.at[half])
                pltpu.sync_copy(lw_hbm.at[rel, sub, g, grp],
                                rbw.at[half])

            for hp, x_hbm in enumerate(xlh):
                @plsc.parallel_loop(0, _AR, unroll=4)
                def _(r):
                    for k in range(_HW // 16):
                        acc[r, pl.ds(16 * k, 16)] = zf

                def sanitize_and_issue(t, buf):
                    idx, wsn, dsn, gbuf, sem = buf
                    ja = lax.shift_right_logical(t, 3) & 1
                    jb = t & 7
                    for k in range(_BS // 16):
                        posv = t * _BS + 16 * k + lane
                        valid = posv < cnt
                        sv = rbs[ja, jb, pl.ds(16 * k, 16)]
                        dvv = rbd[ja, jb, pl.ds(16 * k, 16)]
                        wvv = rbw[ja, jb, pl.ds(16 * k, 16)]
                        idx[pl.ds(16 * k, 16)] = jnp.where(
                            valid & (sv >= 0) & (sv < N), sv, 0)
                        dsn[pl.ds(16 * k, 16)] = jnp.where(
                            valid & (dvv >= 0) & (dvv < _SUB), dvv, 0)
                        wsn[pl.ds(16 * k, 16)] = jnp.where(valid, wvv, zf)
                    pltpu.async_copy(x_hbm.at[idx], gbuf, sem)

                def process(j, buf):
                    idx, wsn, dsn, gbuf, sem = buf

                    def srow(r, _):
                        rv = jnp.full((16,), r, jnp.int32)
                        wv = plsc.load_gather(wsn, [rv])
                        dv = plsc.load_gather(dsn, [rv])
                        for k in range(_HW // 16):
                            plsc.addupdate_scatter(
                                acc, [dv, cols[k]],
                                gbuf[r, pl.ds(16 * k, 16)] * wv)
                        if hp == 0:
                            real = (j * _BS + r) < cnt
                            drow = _SUB + lax.shift_right_logical(dv, 7)
                            plsc.addupdate_scatter(
                                acc, [drow, dv & 127], onef,
                                mask=(lane == 0) & real)
                        return 0
                    lax.fori_loop(0, _BS, srow, 0, unroll=4)

                @pl.when(nb > 0)
                def _():
                    readback(0)
                    sanitize_and_issue(0, bufs[0])

                @pl.when(nb > 1)
                def _():
                    sanitize_and_issue(1, bufs[1])

                @pl.when(nb > 2)
                def _():
                    sanitize_and_issue(2, bufs[2])

                def triple(jj, _):
                    for b in range(3):
                        j = 3 * jj + b
                        buf = bufs[b]

                        @pl.when(j < nb)
                        def _():
                            idx, wsn, dsn, gbuf, sem = buf
                            pltpu.make_async_copy(
                                x_hbm.at[pl.ds(0, _BS)], gbuf, sem).wait()
                            process(j, buf)
                            t = j + 3

                            @pl.when(t < nb)
                            def _():
                                @pl.when((t & 7) == 0)
                                def _():
                                    readback(t)
                                sanitize_and_issue(t, buf)
                    return 0
                lax.fori_loop(0, (nb + 2) // 3, triple, 0)

                pltpu.sync_copy(acc, out_hbm.at[rel, sub, hp, g])
            return 0
        lax.fori_loop(0, 2, subbody, 0)


def _make_scan():
    mesh = plsc.VectorSubcoreMesh(core_axis_name="c", subcore_axis_name="s",
                                  num_cores=_NC, num_subcores=_NT)
    return pl.kernel(
        _scan_body,
        out_type=(jax.ShapeDtypeStruct((3, 2, _NW, _NGRP, 8, _BS), jnp.int32),
                  jax.ShapeDtypeStruct((3, 2, _NW, _NGRP, 8, _BS), jnp.int32),
                  jax.ShapeDtypeStruct((3, 2, _NW, _NGRP, 8, _BS),
                                       jnp.float32),
                  jax.ShapeDtypeStruct((3, 2, _NW, 1, 16), jnp.int32)),
        mesh=mesh,
        compiler_params=pltpu.CompilerParams(needs_layout_passes=False),
        scratch_types=[
            pltpu.VMEM((2, _SSTG), jnp.int32),
            pltpu.VMEM((2, _SSTG), jnp.int32),
            pltpu.VMEM((_SSTG,), jnp.float32),
            pltpu.VMEM((_SSTG,), jnp.float32),
            pltpu.VMEM((_RING, _BS), jnp.int32),
            pltpu.VMEM((_RING, _BS), jnp.int32),
            pltpu.VMEM((_RING, _BS), jnp.float32),
            pltpu.VMEM((_RING, _BS), jnp.int32),
            pltpu.VMEM((_RING, _BS), jnp.int32),
            pltpu.VMEM((_RING, _BS), jnp.float32),
            pltpu.VMEM((1, 16), jnp.int32),
            pltpu.SemaphoreType.DMA,
            pltpu.SemaphoreType.DMA,
        ],
    )


def _make_aggB():
    mesh = plsc.VectorSubcoreMesh(core_axis_name="c", subcore_axis_name="s",
                                  num_cores=_NC, num_subcores=_NT)
    return pl.kernel(
        _agg_body,
        out_type=jax.ShapeDtypeStruct((3, 2, 2, _NW, _AR, _HW), jnp.float32),
        mesh=mesh,
        compiler_params=pltpu.CompilerParams(needs_layout_passes=False),
        scratch_types=[
            pltpu.VMEM((2, 8, _BS), jnp.int32),
            pltpu.VMEM((2, 8, _BS), jnp.int32),
            pltpu.VMEM((2, 8, _BS), jnp.float32),
            pltpu.VMEM((_BS,), jnp.int32),
            pltpu.VMEM((_BS,), jnp.int32),
            pltpu.VMEM((_BS,), jnp.int32),
            pltpu.VMEM((_BS,), jnp.float32),
            pltpu.VMEM((_BS,), jnp.float32),
            pltpu.VMEM((_BS,), jnp.float32),
            pltpu.VMEM((_BS,), jnp.int32),
            pltpu.VMEM((_BS,), jnp.int32),
            pltpu.VMEM((_BS,), jnp.int32),
            pltpu.VMEM((_BS, _HW), jnp.float32),
            pltpu.VMEM((_BS, _HW), jnp.float32),
            pltpu.VMEM((_BS, _HW), jnp.float32),
            pltpu.VMEM((_AR, _HW), jnp.float32),
            pltpu.VMEM((1, 16), jnp.int32),
            pltpu.SemaphoreType.DMA,
            pltpu.SemaphoreType.DMA,
            pltpu.SemaphoreType.DMA,
        ],
    )


_scan = _make_scan()
_aggB = _make_aggB()


def _route(ei_list, w_list):
    return _scan(ei_list[0], w_list[0], ei_list[1], w_list[1],
                 ei_list[2], w_list[2])


def _sc_agg3(x_list, lists):
    ls, ld, lw, cnts = lists
    xs = []
    for x in x_list:
        xs += [x[:, :_HW], x[:, _HW:]]
    out = _aggB(*xs, ls, ld, lw, cnts)
    aggs, degs = [], []
    for r in range(3):
        a = out[r, :, :, :, :_SUB, :]
        a = jnp.transpose(a, (2, 0, 3, 1, 4))
        aggs.append(a.reshape(_NPAD, D)[:N])
        d = out[r, :, 0, :, _SUB:_SUB + 2, :]
        d = jnp.transpose(d, (1, 0, 2, 3))
        d = d.reshape(_NW, 2, 256)[:, :, :_SUB]
        degs.append(d.reshape(_NPAD)[:N])
    return aggs, degs


def kernel(x_acoustic, x_word, edge_index_sim_tic, weight_sim_tic, Ws1_sim_tic, Wn1_sim_tic, b1_sim_tic, Ws2_sim_tic, Wn2_sim_tic, b2_sim_tic, edge_index_sim_w, weight_sim_w, Ws1_sim_w, Wn1_sim_w, b1_sim_w, Ws2_sim_w, Wn2_sim_w, b2_sim_w, edge_index_related_to, weight_related_to, Ws1_related_to, Wn1_related_to, b1_related_to, Ws2_related_to, Wn2_related_to, b2_related_to):
    lists = _route((edge_index_sim_tic, edge_index_sim_w,
                    edge_index_related_to),
                   (weight_sim_tic, weight_sim_w, weight_related_to))
    (a1_t, a1_w, a1_r), (deg_t, deg_w, deg_r) = _sc_agg3(
        (x_acoustic, x_word, x_acoustic), lists)

    inv_t = 1.0 / jnp.clip(deg_t, 1.0, None)
    inv_w = 1.0 / jnp.clip(deg_w, 1.0, None)
    inv_r = 1.0 / jnp.clip(deg_r, 1.0, None)

    hn_t = a1_t * inv_t[:, None]
    hn_w = a1_w * inv_w[:, None]
    hn_r = a1_r * inv_r[:, None]

    pre_ac = _linear(x_acoustic, Ws1_sim_tic, b1_sim_tic)
    pre_w = _linear(x_word, 0.5 * (Ws1_sim_w + Ws1_related_to),
                    0.5 * (b1_sim_w + b1_related_to))
    h_ac = _linear(hn_t, Wn1_sim_tic, jnp.zeros((H,), jnp.float32),
                   add=pre_ac, relu=True)
    h_w = _linear(jnp.concatenate([hn_w, hn_r], axis=1),
                  jnp.concatenate([0.5 * Wn1_sim_w,
                                   0.5 * Wn1_related_to], axis=0),
                  jnp.zeros((H,), jnp.float32), add=pre_w, relu=True)

    y_t = _linear(h_ac, Wn2_sim_tic, jnp.zeros((O,), jnp.float32))
    y_w = _linear(h_w, Wn2_sim_w, jnp.zeros((O,), jnp.float32))
    y_r = _linear(h_ac, Wn2_related_to, jnp.zeros((O,), jnp.float32))

    (a2_t, a2_w, a2_r), _ = _sc_agg3((y_t, y_w, y_r), lists)

    hn2_t = a2_t * inv_t[:, None]
    hn2_w = a2_w * inv_w[:, None]
    hn2_r = a2_r * inv_r[:, None]

    pre2_ac = _linear(h_ac, Ws2_sim_tic, b2_sim_tic)
    pre2_w = _linear(h_w, 0.5 * (Ws2_sim_w + Ws2_related_to),
                     0.5 * (b2_sim_w + b2_related_to))
    o_ac = pre2_ac + hn2_t
    o_w = pre2_w + 0.5 * (hn2_w + hn2_r)
    return (o_ac, o_w)


# speedup vs baseline: 1.8619x; 1.0088x over previous
import functools

import jax
import jax.numpy as jnp
from jax import lax
from jax.experimental import pallas as pl
from jax.experimental.pallas import tpu as pltpu
from jax.experimental.pallas import tpu_sc as plsc

N = 10000
E = 160000
D = 256
H = 512
O = 256

_NC = 2
_NT = 16
_HALF = N // _NC
_ET = E // _NT
_STG = 640
_NB = -(-_ET // 128)
_ZROW = 320
_DR = 48

_ROW_BLK = 1000


def _linear_body(x_ref, w_ref, b_ref, add_ref, o_ref, *, relu):
    acc = jnp.dot(x_ref[...], w_ref[...], preferred_element_type=jnp.float32)
    acc = acc + b_ref[...]
    if add_ref is not None:
        acc = acc + add_ref[...]
    if relu:
        acc = jnp.maximum(acc, 0.0)
    o_ref[...] = acc


def _linear(x, w, b, add=None, relu=False):
    n, k = x.shape
    f = w.shape[1]
    grid = (n // _ROW_BLK,)
    in_specs = [
        pl.BlockSpec((_ROW_BLK, k), lambda i: (i, 0)),
        pl.BlockSpec((k, f), lambda i: (0, 0)),
        pl.BlockSpec((1, f), lambda i: (0, 0)),
    ]
    args = [x, w, b.reshape(1, f)]
    if add is not None:
        in_specs.append(pl.BlockSpec((_ROW_BLK, f), lambda i: (i, 0)))
        args.append(add)
        body = functools.partial(_linear_body, relu=relu)
    else:
        body = lambda x_ref, w_ref, b_ref, o_ref: _linear_body(
            x_ref, w_ref, b_ref, None, o_ref, relu=relu)
    return pl.pallas_call(
        body,
        grid=grid,
        in_specs=in_specs,
        out_specs=pl.BlockSpec((_ROW_BLK, f), lambda i: (i, 0)),
        out_shape=jax.ShapeDtypeStruct((n, f), jnp.float32),
    )(*args)


_NW = _NC * _NT
_TROW = 320
_SUB = 160
_NPAD = _NW * _TROW
_BS = 64
_RING = 32
_SSTG = 1280
_NGRP = 314
_HW = 128
_AR = _SUB + 8


def _scan_body(ei0, w0, ei1, w1, ei2, w2,
               ls_hbm, ld_hbm, lw_hbm, cnt_hbm,
               sd0, sd1, ew0, ew1, rs0, rd0, rw0, rs1, rd1, rw1,
               cntbuf, sem0, sem1):
    c = lax.axis_index("c")
    s = lax.axis_index("s")
    g = c * _NT + s
    base = g * _TROW
    rings = ((rs0, rd0, rw0), (rs1, rd1, rw1))

    for rel, (ei, w_hbm) in enumerate(((ei0, w0), (ei1, w1), (ei2, w2))):
        def issue(h, sd, ew, sem):
            pltpu.async_copy(ei.at[:, pl.ds(h * _SSTG, _SSTG)], sd, sem)
            pltpu.async_copy(w_hbm.at[pl.ds(h * _SSTG, _SSTG)], ew, sem)

        def wait(sd, ew, sem):
            pltpu.make_async_copy(ei.at[:, pl.ds(0, _SSTG)], sd, sem).wait()
            pltpu.make_async_copy(w_hbm.at[pl.ds(0, _SSTG)], ew, sem).wait()

        def filt_stage(sd, ew, carry):
            def filt(i, carry):
                c0, c1, f0, f1 = carry
                sv = sd[0, pl.ds(i * 16, 16)]
                dv = sd[1, pl.ds(i * 16, 16)]
                wv = ew[pl.ds(i * 16, 16)]
                ld = dv - base
                cs = []
                for sub, cc in ((0, c0), (1, c1)):
                    lds = ld - sub * _SUB
                    m = (lds >= 0) & (lds < _SUB)
                    mi = jnp.where(m, 1, 0)
                    pos = cc + plsc.cumsum(mi) - 1
                    row = lax.shift_right_logical(pos, 6) & (_RING - 1)
                    col = pos & (_BS - 1)
                    rs, rd, rw = rings[sub]
                    plsc.store_scatter(rs, [row, col], sv, mask=m)
                    plsc.store_scatter(rd, [row, col], lds, mask=m)
                    plsc.store_scatter(rw, [row, col], wv, mask=m)
                    cs.append(cc + jnp.sum(mi))
                return cs[0], cs[1], f0, f1
            def filt_pl(i, carry):
                return filt(i, carry)
            return plsc.parallel_loop(0, _SSTG // 16, unroll=4,
                                      carry=carry)(filt_pl)

        def flush(sub, cc, fl):
            rs, rd, rw = rings[sub]

            def grp(j8, _):
                rr = (j8 & 3) * 8
                pltpu.sync_copy(rs.at[pl.ds(rr, 8)],
                                ls_hbm.at[rel, sub, g, j8])
                pltpu.sync_copy(rd.at[pl.ds(rr, 8)],
                                ld_hbm.at[rel, sub, g, j8])
                pltpu.sync_copy(rw.at[pl.ds(rr, 8)],
                                lw_hbm.at[rel, sub, g, j8])
                return 0
            lax.fori_loop(fl, cc, grp, 0)
            return cc

        def stage_pair(hh, carry):
            h0 = 2 * hh
            wait(sd0, ew0, sem0)
            issue(h0 + 1, sd1, ew1, sem1)
            carry = filt_stage(sd0, ew0, carry)
            c0, c1, f0, f1 = carry
            f0 = flush(0, lax.shift_right_logical(c0, 9), f0)
            f1 = flush(1, lax.shift_right_logical(c1, 9), f1)
            wait(sd1, ew1, sem1)
            issue(h0 + 2, sd0, ew0, sem0)
            carry = filt_stage(sd1, ew1, (c0, c1, f0, f1))
            c0, c1, f0, f1 = carry
            f0 = flush(0, lax.shift_right_logical(c0, 9), f0)
            f1 = flush(1, lax.shift_right_logical(c1, 9), f1)
            return c0, c1, f0, f1

        issue(0, sd0, ew0, sem0)
        nst = E // _SSTG
        carry = lax.fori_loop(0, (nst - 1) // 2, stage_pair,
                              (jnp.int32(0), jnp.int32(0),
                               jnp.int32(0), jnp.int32(0)))
        wait(sd0, ew0, sem0)
        c0, c1, f0, f1 = filt_stage(sd0, ew0, carry)

        for sub, cc, fl in ((0, c0, f0), (1, c1, f1)):
            ng = lax.shift_right_logical(cc + 511, 9)
            flush(sub, ng, fl)
            cntbuf[0, pl.ds(0, 16)] = jnp.full((16,), cc, jnp.int32)
            pltpu.sync_copy(cntbuf, cnt_hbm.at[rel, sub, g])


def _agg_body(x0l, x0h, x1l, x1h, x2l, x2h,
              ls_hbm, ld_hbm, lw_hbm, cnt_hbm, out_hbm,
              rbs, rbd, rbw, idxa, idxb, idxc, wsa, wsb, wsc,
              dsa, dsb, dsc, gbufa, gbufb, gbufc, acc, cntv,
              sema, semb, semc):
    c = lax.axis_index("c")
    s = lax.axis_index("s")
    g = c * _NT + s
    lane = lax.iota(jnp.int32, 16)
    zf = jnp.zeros((16,), jnp.float32)
    zc = jnp.zeros((16,), jnp.int32)
    onef = jnp.ones((16,), jnp.float32)
    cols = [lane + 16 * k for k in range(_HW // 16)]
    bufs = ((idxa, wsa, dsa, gbufa, sema), (idxb, wsb, dsb, gbufb, semb),
            (idxc, wsc, dsc, gbufc, semc))

    for rel, xlh in enumerate(((x0l, x0h), (x1l, x1h), (x2l, x2h))):
        def subbody(sub, _):
            pltpu.sync_copy(cnt_hbm.at[rel, sub, g], cntv)
            cnt = jnp.sum(jnp.where(lane == 0, cntv[0, pl.ds(0, 16)], zc))
            nb = lax.shift_right_logical(cnt + _BS - 1, 6)

            def readback(t):
                half = lax.shift_right_logical(t, 3) & 1
                grp = lax.shift_right_logical(t, 3)
                pltpu.sync_copy(ls_hbm.at[rel, sub, g, grp],
                                rbs.at[half])
                pltpu.sync_copy(ld_hbm.at[rel, sub, g, grp],
                                rbd.at[half])
                pltpu.sync_copy(lw_hbm.at[rel, sub, g, grp],
                                rbw.at[half])

            for hp, x_hbm in enumerate(xlh):
                @plsc.parallel_loop(0, _AR, unroll=4)
                def _(r):
                    for k in range(_HW // 16):
                        acc[r, pl.ds(16 * k, 16)] = zf

                def sanitize_and_issue(t, buf):
                    idx, wsn, dsn, gbuf, sem = buf
                    ja = lax.shift_right_logical(t, 3) & 1
                    jb = t & 7
                    for k in range(_BS // 16):
                        posv = t * _BS + 16 * k + lane
                        valid = posv < cnt
                        sv = rbs[ja, jb, pl.ds(16 * k, 16)]
                        dvv = rbd[ja, jb, pl.ds(16 * k, 16)]
                        wvv = rbw[ja, jb, pl.ds(16 * k, 16)]
                        idx[pl.ds(16 * k, 16)] = jnp.where(
                            valid & (sv >= 0) & (sv < N), sv, 0)
                        dsn[pl.ds(16 * k, 16)] = jnp.where(
                            valid & (dvv >= 0) & (dvv < _SUB), dvv, 0)
                        wsn[pl.ds(16 * k, 16)] = jnp.where(valid, wvv, zf)
                    pltpu.async_copy(x_hbm.at[idx], gbuf, sem)

                def process(j, buf):
                    idx, wsn, dsn, gbuf, sem = buf

                    def srow(r, _):
                        rv = jnp.full((16,), r, jnp.int32)
                        wv = plsc.load_gather(wsn, [rv])
                        dv = plsc.load_gather(dsn, [rv])
                        for k in range(_HW // 16):
                            plsc.addupdate_scatter(
                                acc, [dv, cols[k]],
                                gbuf[r, pl.ds(16 * k, 16)] * wv)
                        if hp == 0:
                            real = (j * _BS + r) < cnt
                            drow = _SUB + lax.shift_right_logical(dv, 7)
                            plsc.addupdate_scatter(
                                acc, [drow, dv & 127], onef,
                                mask=(lane == 0) & real)
                        return 0
                    lax.fori_loop(0, _BS, srow, 0, unroll=8)

                @pl.when(nb > 0)
                def _():
                    readback(0)
                    sanitize_and_issue(0, bufs[0])

                @pl.when(nb > 1)
                def _():
                    sanitize_and_issue(1, bufs[1])

                @pl.when(nb > 2)
                def _():
                    sanitize_and_issue(2, bufs[2])

                def triple(jj, _):
                    for b in range(3):
                        j = 3 * jj + b
                        buf = bufs[b]

                        @pl.when(j < nb)
                        def _():
                            idx, wsn, dsn, gbuf, sem = buf
                            pltpu.make_async_copy(
                                x_hbm.at[pl.ds(0, _BS)], gbuf, sem).wait()
                            process(j, buf)
                            t = j + 3

                            @pl.when(t < nb)
                            def _():
                                @pl.when((t & 7) == 0)
                                def _():
                                    readback(t)
                                sanitize_and_issue(t, buf)
                    return 0
                lax.fori_loop(0, (nb + 2) // 3, triple, 0)

                pltpu.sync_copy(acc, out_hbm.at[rel, sub, hp, g])
            return 0
        lax.fori_loop(0, 2, subbody, 0)


def _make_scan():
    mesh = plsc.VectorSubcoreMesh(core_axis_name="c", subcore_axis_name="s",
                                  num_cores=_NC, num_subcores=_NT)
    return pl.kernel(
        _scan_body,
        out_type=(jax.ShapeDtypeStruct((3, 2, _NW, _NGRP, 8, _BS), jnp.int32),
                  jax.ShapeDtypeStruct((3, 2, _NW, _NGRP, 8, _BS), jnp.int32),
                  jax.ShapeDtypeStruct((3, 2, _NW, _NGRP, 8, _BS),
                                       jnp.float32),
                  jax.ShapeDtypeStruct((3, 2, _NW, 1, 16), jnp.int32)),
        mesh=mesh,
        compiler_params=pltpu.CompilerParams(needs_layout_passes=False),
        scratch_types=[
            pltpu.VMEM((2, _SSTG), jnp.int32),
            pltpu.VMEM((2, _SSTG), jnp.int32),
            pltpu.VMEM((_SSTG,), jnp.float32),
            pltpu.VMEM((_SSTG,), jnp.float32),
            pltpu.VMEM((_RING, _BS), jnp.int32),
            pltpu.VMEM((_RING, _BS), jnp.int32),
            pltpu.VMEM((_RING, _BS), jnp.float32),
            pltpu.VMEM((_RING, _BS), jnp.int32),
            pltpu.VMEM((_RING, _BS), jnp.int32),
            pltpu.VMEM((_RING, _BS), jnp.float32),
            pltpu.VMEM((1, 16), jnp.int32),
            pltpu.SemaphoreType.DMA,
            pltpu.SemaphoreType.DMA,
        ],
    )


def _make_aggB():
    mesh = plsc.VectorSubcoreMesh(core_axis_name="c", subcore_axis_name="s",
                                  num_cores=_NC, num_subcores=_NT)
    return pl.kernel(
        _agg_body,
        out_type=jax.ShapeDtypeStruct((3, 2, 2, _NW, _AR, _HW), jnp.float32),
        mesh=mesh,
        compiler_params=pltpu.CompilerParams(needs_layout_passes=False),
        scratch_types=[
            pltpu.VMEM((2, 8, _BS), jnp.int32),
            pltpu.VMEM((2, 8, _BS), jnp.int32),
            pltpu.VMEM((2, 8, _BS), jnp.float32),
            pltpu.VMEM((_BS,), jnp.int32),
            pltpu.VMEM((_BS,), jnp.int32),
            pltpu.VMEM((_BS,), jnp.int32),
            pltpu.VMEM((_BS,), jnp.float32),
            pltpu.VMEM((_BS,), jnp.float32),
            pltpu.VMEM((_BS,), jnp.float32),
            pltpu.VMEM((_BS,), jnp.int32),
            pltpu.VMEM((_BS,), jnp.int32),
            pltpu.VMEM((_BS,), jnp.int32),
            pltpu.VMEM((_BS, _HW), jnp.float32),
            pltpu.VMEM((_BS, _HW), jnp.float32),
            pltpu.VMEM((_BS, _HW), jnp.float32),
            pltpu.VMEM((_AR, _HW), jnp.float32),
            pltpu.VMEM((1, 16), jnp.int32),
            pltpu.SemaphoreType.DMA,
            pltpu.SemaphoreType.DMA,
            pltpu.SemaphoreType.DMA,
        ],
    )


_scan = _make_scan()
_aggB = _make_aggB()


def _route(ei_list, w_list):
    return _scan(ei_list[0], w_list[0], ei_list[1], w_list[1],
                 ei_list[2], w_list[2])


def _sc_agg3(x_list, lists):
    ls, ld, lw, cnts = lists
    xs = []
    for x in x_list:
        xs += [x[:, :_HW], x[:, _HW:]]
    out = _aggB(*xs, ls, ld, lw, cnts)
    aggs, degs = [], []
    for r in range(3):
        a = out[r, :, :, :, :_SUB, :]
        a = jnp.transpose(a, (2, 0, 3, 1, 4))
        aggs.append(a.reshape(_NPAD, D)[:N])
        d = out[r, :, 0, :, _SUB:_SUB + 2, :]
        d = jnp.transpose(d, (1, 0, 2, 3))
        d = d.reshape(_NW, 2, 256)[:, :, :_SUB]
        degs.append(d.reshape(_NPAD)[:N])
    return aggs, degs


def kernel(x_acoustic, x_word, edge_index_sim_tic, weight_sim_tic, Ws1_sim_tic, Wn1_sim_tic, b1_sim_tic, Ws2_sim_tic, Wn2_sim_tic, b2_sim_tic, edge_index_sim_w, weight_sim_w, Ws1_sim_w, Wn1_sim_w, b1_sim_w, Ws2_sim_w, Wn2_sim_w, b2_sim_w, edge_index_related_to, weight_related_to, Ws1_related_to, Wn1_related_to, b1_related_to, Ws2_related_to, Wn2_related_to, b2_related_to):
    lists = _route((edge_index_sim_tic, edge_index_sim_w,
                    edge_index_related_to),
                   (weight_sim_tic, weight_sim_w, weight_related_to))
    (a1_t, a1_w, a1_r), (deg_t, deg_w, deg_r) = _sc_agg3(
        (x_acoustic, x_word, x_acoustic), lists)

    inv_t = 1.0 / jnp.clip(deg_t, 1.0, None)
    inv_w = 1.0 / jnp.clip(deg_w, 1.0, None)
    inv_r = 1.0 / jnp.clip(deg_r, 1.0, None)

    hn_t = a1_t * inv_t[:, None]
    hn_w = a1_w * inv_w[:, None]
    hn_r = a1_r * inv_r[:, None]

    pre_ac = _linear(x_acoustic, Ws1_sim_tic, b1_sim_tic)
    pre_w = _linear(x_word, 0.5 * (Ws1_sim_w + Ws1_related_to),
                    0.5 * (b1_sim_w + b1_related_to))
    h_ac = _linear(hn_t, Wn1_sim_tic, jnp.zeros((H,), jnp.float32),
                   add=pre_ac, relu=True)
    h_w = _linear(jnp.concatenate([hn_w, hn_r], axis=1),
                  jnp.concatenate([0.5 * Wn1_sim_w,
                                   0.5 * Wn1_related_to], axis=0),
                  jnp.zeros((H,), jnp.float32), add=pre_w, relu=True)

    y_t = _linear(h_ac, Wn2_sim_tic, jnp.zeros((O,), jnp.float32))
    y_w = _linear(h_w, Wn2_sim_w, jnp.zeros((O,), jnp.float32))
    y_r = _linear(h_ac, Wn2_related_to, jnp.zeros((O,), jnp.float32))

    (a2_t, a2_w, a2_r), _ = _sc_agg3((y_t, y_w, y_r), lists)

    hn2_t = a2_t * inv_t[:, None]
    hn2_w = a2_w * inv_w[:, None]
    hn2_r = a2_r * inv_r[:, None]

    pre2_ac = _linear(h_ac, Ws2_sim_tic, b2_sim_tic)
    pre2_w = _linear(h_w, 0.5 * (Ws2_sim_w + Ws2_related_to),
                     0.5 * (b2_sim_w + b2_related_to))
    o_ac = pre2_ac + hn2_t
    o_w = pre2_w + 0.5 * (hn2_w + hn2_r)
    return (o_ac, o_w)
